# baseline (speedup 1.0000x reference)
import jax
import jax.numpy as jnp
from jax.experimental import pallas as pl
from jax.experimental.pallas import tpu as pltpu

_B = 8
_N = 4096
_D = 512
_C = 1024
_NU = 0.001
_K = 1
_TB = 2048
_NBLK = (_B * _N) // _TB
_LOSS_SCALE = 1.0 / (_NU * _B * _N * _K)


def _dist_kernel(e_ref, ct_ref, r_ref, score_ref, loss_ref, eout_ref,
                 ctm2_ref, cen_ref):
    i = pl.program_id(0)

    @pl.when(i == 0)
    def _prep():
        c = ct_ref[...]
        ct = jnp.transpose(c)
        ctm2_ref[...] = (-2.0 * ct).astype(jnp.bfloat16)
        cen_ref[...] = jnp.sum(ct * ct, axis=0, keepdims=True)

    e = e_ref[...]
    p = jnp.dot(e.astype(jnp.bfloat16), ctm2_ref[...],
                preferred_element_type=jnp.float32)
    d2 = cen_ref[...] + p
    m = jnp.min(d2, axis=1, keepdims=True)
    feat = jnp.sum(e * e, axis=1, keepdims=True)
    dist = jnp.sqrt(feat + m)
    score_ref[...] = dist.reshape(1, 1, 32, 64)
    eout_ref[...] = e
    part = jnp.sum(jnp.maximum(dist - r_ref[0] * r_ref[0], 0.0))

    @pl.when(i == 0)
    def _init():
        loss_ref[0, 0] = 0.0

    loss_ref[0, 0] += part

    @pl.when(i == _NBLK - 1)
    def _finish():
        loss_ref[0, 0] = loss_ref[0, 0] * _LOSS_SCALE


def kernel(embeds, centroids, r):
    e2d = embeds.reshape(_B * _N, _D)
    score_flat, loss, e_out = pl.pallas_call(
        _dist_kernel,
        grid=(_NBLK,),
        in_specs=[
            pl.BlockSpec((_TB, _D), lambda i: (i, 0)),
            pl.BlockSpec((_C, _D), lambda i: (0, 0)),
            pl.BlockSpec(memory_space=pltpu.SMEM),
        ],
        out_specs=[
            pl.BlockSpec((1, 1, 32, 64), lambda i: (i // 2, 0, i % 2, 0)),
            pl.BlockSpec(memory_space=pltpu.SMEM),
            pl.BlockSpec((_TB, _D), lambda i: (i, 0)),
        ],
        out_shape=[
            jax.ShapeDtypeStruct((_B, 1, 64, 64), jnp.float32),
            jax.ShapeDtypeStruct((1, 1), jnp.float32),
            jax.ShapeDtypeStruct((_B * _N, _D), jnp.float32),
        ],
        scratch_shapes=[
            pltpu.VMEM((_D, _C), jnp.bfloat16),
            pltpu.VMEM((1, _C), jnp.float32),
        ],
        compiler_params=pltpu.CompilerParams(
            dimension_semantics=("arbitrary",),
        ),
    )(e2d, centroids, r)
    return (loss[0, 0], score_flat, e_out.reshape(_B, _N, _D))


# speedup vs baseline: 1.0306x; 1.0306x over previous
import jax
import jax.numpy as jnp
from jax.experimental import pallas as pl
from jax.experimental.pallas import tpu as pltpu

_B = 8
_N = 4096
_D = 512
_C = 1024
_NU = 0.001
_K = 1
_TB = 4096
_NBLK = (_B * _N) // _TB
_LOSS_SCALE = 1.0 / (_NU * _B * _N * _K)


def _dist_kernel(e_ref, ct_ref, r_ref, score_ref, loss_ref, eout_ref,
                 ctm2_ref, cen_ref):
    i = pl.program_id(0)

    @pl.when(i == 0)
    def _prep():
        c = ct_ref[...]
        ct = jnp.transpose(c)
        ctm2_ref[...] = (-2.0 * ct).astype(jnp.bfloat16)
        cen_ref[...] = jnp.sum(ct * ct, axis=0, keepdims=True)

    e = e_ref[...]
    p = jnp.dot(e.astype(jnp.bfloat16), ctm2_ref[...],
                preferred_element_type=jnp.float32)
    d2 = cen_ref[...] + p
    m = jnp.min(d2, axis=1, keepdims=True)
    feat = jnp.sum(e * e, axis=1, keepdims=True)
    dist = jnp.sqrt(feat + m)
    score_ref[...] = dist.reshape(1, 1, 64, 64)
    eout_ref[...] = e
    part = jnp.sum(jnp.maximum(dist - r_ref[0] * r_ref[0], 0.0))

    @pl.when(i == 0)
    def _init():
        loss_ref[0, 0] = 0.0

    loss_ref[0, 0] += part

    @pl.when(i == _NBLK - 1)
    def _finish():
        loss_ref[0, 0] = loss_ref[0, 0] * _LOSS_SCALE


def kernel(embeds, centroids, r):
    e2d = embeds.reshape(_B * _N, _D)
    score_flat, loss, e_out = pl.pallas_call(
        _dist_kernel,
        grid=(_NBLK,),
        in_specs=[
            pl.BlockSpec((_TB, _D), lambda i: (i, 0)),
            pl.BlockSpec((_C, _D), lambda i: (0, 0)),
            pl.BlockSpec(memory_space=pltpu.SMEM),
        ],
        out_specs=[
            pl.BlockSpec((1, 1, 64, 64), lambda i: (i, 0, 0, 0)),
            pl.BlockSpec(memory_space=pltpu.SMEM),
            pl.BlockSpec((_TB, _D), lambda i: (i, 0)),
        ],
        out_shape=[
            jax.ShapeDtypeStruct((_B, 1, 64, 64), jnp.float32),
            jax.ShapeDtypeStruct((1, 1), jnp.float32),
            jax.ShapeDtypeStruct((_B * _N, _D), jnp.float32),
        ],
        scratch_shapes=[
            pltpu.VMEM((_D, _C), jnp.bfloat16),
            pltpu.VMEM((1, _C), jnp.float32),
        ],
        compiler_params=pltpu.CompilerParams(
            dimension_semantics=("arbitrary",),
        ),
    )(e2d, centroids, r)
    return (loss[0, 0], score_flat, e_out.reshape(_B, _N, _D))
